# Initial kernel scaffold; baseline (speedup 1.0000x reference)
import jax
import jax.numpy as jnp
from jax.experimental import pallas as pl


def kernel(x_rows, x_cols, x_vals, adj_rows, adj_cols, adj_vals, kernel):
    raise NotImplementedError("write your pallas kernel here")


# speedup vs baseline: 7.0744x; 7.0744x over previous
import dataclasses
import functools

import jax
import jax.numpy as jnp
from jax import lax
from jax.experimental import pallas as pl
from jax.experimental.pallas import tpu as pltpu
from jax.experimental.pallas import tpu_sc as plsc

N = 10000
D = 128
OUT = 32
NNZ = 320000
ROWS = NNZ // 128
NC = 2
NS = 16
NW = NC * NS
FULL_T = ROWS // NW
REM = ROWS - FULL_T * NW
ND = N * D
ZW = 16000

_mesh = plsc.VectorSubcoreMesh(core_axis_name="c", subcore_axis_name="s")

_cp = pltpu.CompilerParams()
for _f, _v in (("needs_layout_passes", False), ("use_tc_tiling_on_sc", False)):
    if _f in pltpu.CompilerParams.__dataclass_fields__:
        _cp = dataclasses.replace(_cp, **{_f: _v})


def _zero16():
    return jnp.zeros((16,), jnp.float32)


@functools.partial(
    pl.kernel,
    mesh=_mesh,
    out_type=jax.ShapeDtypeStruct((NC * ND,), jnp.float32),
    scratch_types=[
        pltpu.VMEM((128,), jnp.int32),
        pltpu.VMEM((128,), jnp.int32),
        pltpu.VMEM((128,), jnp.float32),
        pltpu.VMEM((ZW,), jnp.float32),
        pltpu.VMEM_SHARED((ND,), jnp.float32),
    ],
)
def _stage1(xr_hbm, xc_hbm, xv_hbm, out_hbm, r_v, c_v, v_v, zb, xd_sh):
    cid = lax.axis_index("c")
    sid = lax.axis_index("s")
    gwid = sid * NC + cid

    @pl.loop(0, ZW, step=16)
    def _(i):
        zb[pl.ds(i, 16)] = _zero16()

    @pl.loop(0, 5)
    def _(t):
        pltpu.sync_copy(zb, xd_sh.at[pl.ds(sid * (5 * ZW) + t * ZW, ZW)])

    plsc.subcore_barrier()

    def do_chunk(chunk):
        base = chunk * 128
        pltpu.sync_copy(xr_hbm.at[pl.ds(base, 128)], r_v)
        pltpu.sync_copy(xc_hbm.at[pl.ds(base, 128)], c_v)
        pltpu.sync_copy(xv_hbm.at[pl.ds(base, 128)], v_v)

        @pl.loop(0, 128, step=16)
        def _(i):
            c_v[pl.ds(i, 16)] = r_v[pl.ds(i, 16)] * D + c_v[pl.ds(i, 16)]

        pltpu.sync_copy(v_v, xd_sh.at[c_v], add=True)

    @pl.loop(0, FULL_T)
    def _(t):
        do_chunk(t * NW + gwid)

    @pl.when(gwid < REM)
    def _():
        do_chunk(FULL_T * NW + gwid)

    plsc.subcore_barrier()

    @pl.loop(0, 5)
    def _(t):
        off = sid * (5 * ZW) + t * ZW
        pltpu.sync_copy(xd_sh.at[pl.ds(off, ZW)],
                        out_hbm.at[pl.ds(cid * ND + off, ZW)])


@functools.partial(
    pl.kernel,
    mesh=_mesh,
    compiler_params=_cp,
    out_type=jax.ShapeDtypeStruct((NC * N, OUT), jnp.float32),
    scratch_types=[
        pltpu.VMEM((128,), jnp.int32),
        pltpu.VMEM((128,), jnp.int32),
        pltpu.VMEM((128,), jnp.float32),
        pltpu.VMEM((128, OUT), jnp.float32),
        pltpu.VMEM((640, OUT), jnp.float32),
        pltpu.VMEM_SHARED((N, OUT), jnp.float32),
    ],
)
def _stage2(ar_hbm, ac_hbm, av_hbm, h_hbm, out_hbm,
            r_v, c_v, v_v, buf, zb, y_sh):
    cid = lax.axis_index("c")
    sid = lax.axis_index("s")
    gwid = sid * NC + cid

    @pl.loop(0, 640)
    def _(r):
        zb[r, pl.ds(0, 16)] = _zero16()
        zb[r, pl.ds(16, 16)] = _zero16()

    @pl.when(sid < 15)
    def _():
        pltpu.sync_copy(zb.at[pl.ds(0, 624)], y_sh.at[pl.ds(sid * 624, 624)])

    @pl.when(sid == 15)
    def _():
        pltpu.sync_copy(zb, y_sh.at[pl.ds(15 * 624, 640)])

    plsc.subcore_barrier()

    def do_chunk(chunk):
        base = chunk * 128
        pltpu.sync_copy(ar_hbm.at[pl.ds(base, 128)], r_v)
        pltpu.sync_copy(ac_hbm.at[pl.ds(base, 128)], c_v)
        pltpu.sync_copy(av_hbm.at[pl.ds(base, 128)], v_v)
        pltpu.sync_copy(h_hbm.at[c_v], buf)

        @pl.loop(0, 128)
        def _(k):
            bc = plsc.load_gather(v_v, [jnp.full((16,), k, jnp.int32)])
            buf[k, pl.ds(0, 16)] = buf[k, pl.ds(0, 16)] * bc
            buf[k, pl.ds(16, 16)] = buf[k, pl.ds(16, 16)] * bc

        pltpu.sync_copy(buf, y_sh.at[r_v], add=True)

    @pl.loop(0, FULL_T)
    def _(t):
        do_chunk(t * NW + gwid)

    @pl.when(gwid < REM)
    def _():
        do_chunk(FULL_T * NW + gwid)

    plsc.subcore_barrier()

    @pl.when(sid < 15)
    def _():
        pltpu.sync_copy(y_sh.at[pl.ds(sid * 624, 624)],
                        out_hbm.at[pl.ds(cid * N + sid * 624, 624)])

    @pl.when(sid == 15)
    def _():
        pltpu.sync_copy(y_sh.at[pl.ds(15 * 624, 640)],
                        out_hbm.at[pl.ds(cid * N + 15 * 624, 640)])


def _mm_body(xp_ref, w_ref, h_ref):
    x = xp_ref[0] + xp_ref[1]
    h_ref[...] = jnp.dot(x, w_ref[...], preferred_element_type=jnp.float32)


def _matmul(xp, w):
    bn = 2000
    return pl.pallas_call(
        _mm_body,
        grid=(N // bn,),
        in_specs=[
            pl.BlockSpec((NC, bn, D), lambda i: (0, i, 0)),
            pl.BlockSpec((D, OUT), lambda i: (0, 0)),
        ],
        out_specs=pl.BlockSpec((bn, OUT), lambda i: (i, 0)),
        out_shape=jax.ShapeDtypeStruct((N, OUT), jnp.float32),
    )(xp, w)


def _fin_body(yp_ref, o_ref):
    o_ref[...] = jnp.maximum(yp_ref[0] + yp_ref[1], 0.0)


def _finish(yp):
    bn = 2000
    return pl.pallas_call(
        _fin_body,
        grid=(N // bn,),
        in_specs=[pl.BlockSpec((NC, bn, OUT), lambda i: (0, i, 0))],
        out_specs=pl.BlockSpec((bn, OUT), lambda i: (i, 0)),
        out_shape=jax.ShapeDtypeStruct((N, OUT), jnp.float32),
    )(yp)


def kernel(x_rows, x_cols, x_vals, adj_rows, adj_cols, adj_vals, kernel):
    xr = x_rows.astype(jnp.int32)
    xc = x_cols.astype(jnp.int32)
    ar = adj_rows.astype(jnp.int32)
    ac = adj_cols.astype(jnp.int32)

    xd = _stage1(xr, xc, x_vals)
    h = _matmul(xd.reshape(NC, N, D), kernel)
    yp = _stage2(ar, ac, adj_vals, h)
    return _finish(yp.reshape(NC, N, OUT))


# speedup vs baseline: 19.2672x; 2.7235x over previous
import dataclasses
import functools

import jax
import jax.numpy as jnp
from jax import lax
from jax.experimental import pallas as pl
from jax.experimental.pallas import tpu as pltpu
from jax.experimental.pallas import tpu_sc as plsc

N = 10000
D = 128
OUT = 32
NNZ = 320000
ROWS = NNZ // 128
NC = 2
NS = 16
NW = NC * NS
FULL_T = ROWS // NW
REM = ROWS - FULL_T * NW
ND = N * D
SL1 = ND // NS

_mesh = plsc.VectorSubcoreMesh(core_axis_name="c", subcore_axis_name="s")

_cp = pltpu.CompilerParams()
for _f, _v in (("needs_layout_passes", False), ("use_tc_tiling_on_sc", False)):
    if _f in pltpu.CompilerParams.__dataclass_fields__:
        _cp = dataclasses.replace(_cp, **{_f: _v})


def _worker_ids():
    cid = lax.axis_index("c")
    sid = lax.axis_index("s")
    gwid = sid * NC + cid
    cbase = gwid * FULL_T + jnp.minimum(gwid, REM)
    return cid, sid, gwid, cbase


def _bulk_load(pairs, cbase, gwid):
    @pl.when(gwid < REM)
    def _():
        for hbm, vmem in pairs:
            pltpu.sync_copy(hbm.at[pl.ds(cbase, FULL_T + 1)], vmem)

    @pl.when(gwid >= REM)
    def _():
        for hbm, vmem in pairs:
            pltpu.sync_copy(hbm.at[pl.ds(cbase, FULL_T)],
                            vmem.at[pl.ds(0, FULL_T)])


@functools.partial(
    pl.kernel,
    mesh=_mesh,
    compiler_params=_cp,
    out_type=jax.ShapeDtypeStruct((NC * ND,), jnp.float32),
    scratch_types=[
        pltpu.VMEM((FULL_T + 1, 128), jnp.int32),
        pltpu.VMEM((FULL_T + 1, 128), jnp.float32),
        pltpu.VMEM_SHARED((ND,), jnp.float32),
        pltpu.SemaphoreType.DMA,
        pltpu.SemaphoreType.DMA,
    ],
)
def _stage1(xf_hbm, xv_hbm, z_hbm, out_hbm, idx_all, vals_all, xd_sh, s0, s1):
    cid, sid, gwid, cbase = _worker_ids()

    pltpu.sync_copy(z_hbm.at[pl.ds(sid * SL1, SL1)],
                    xd_sh.at[pl.ds(sid * SL1, SL1)])
    _bulk_load([(xf_hbm, idx_all), (xv_hbm, vals_all)], cbase, gwid)
    plsc.subcore_barrier()

    @pl.loop(0, FULL_T, step=2)
    def _(t):
        d0 = pltpu.async_copy(vals_all.at[t], xd_sh.at[idx_all.at[t]],
                              s0, add=True)
        d0.wait()
        d1 = pltpu.async_copy(vals_all.at[t + 1], xd_sh.at[idx_all.at[t + 1]],
                              s1, add=True)
        d1.wait()

    @pl.when(gwid < REM)
    def _():
        pltpu.sync_copy(vals_all.at[FULL_T], xd_sh.at[idx_all.at[FULL_T]],
                        add=True)

    plsc.subcore_barrier()
    pltpu.sync_copy(xd_sh.at[pl.ds(sid * SL1, SL1)],
                    out_hbm.at[pl.ds(cid * ND + sid * SL1, SL1)])


@functools.partial(
    pl.kernel,
    mesh=_mesh,
    compiler_params=_cp,
    out_type=jax.ShapeDtypeStruct((NC * N, OUT), jnp.float32),
    scratch_types=[
        pltpu.VMEM((FULL_T + 1, 128), jnp.int32),
        pltpu.VMEM((FULL_T + 1, 128), jnp.int32),
        pltpu.VMEM((FULL_T + 1, 128), jnp.float32),
        pltpu.VMEM((2, 128, OUT), jnp.float32),
        pltpu.VMEM_SHARED((N, OUT), jnp.float32),
        pltpu.SemaphoreType.DMA,
        pltpu.SemaphoreType.DMA,
    ],
)
def _stage2(ar_hbm, ac_hbm, av_hbm, h_hbm, z_hbm, out_hbm,
            rows_all, cols_all, vals_all, bufs, y_sh, g0, g1):
    cid, sid, gwid, cbase = _worker_ids()

    @pl.when(sid < 15)
    def _():
        pltpu.sync_copy(z_hbm.at[pl.ds(sid * 624, 624)],
                        y_sh.at[pl.ds(sid * 624, 624)])

    @pl.when(sid == 15)
    def _():
        pltpu.sync_copy(z_hbm.at[pl.ds(15 * 624, 640)],
                        y_sh.at[pl.ds(15 * 624, 640)])

    _bulk_load([(ar_hbm, rows_all), (ac_hbm, cols_all), (av_hbm, vals_all)],
               cbase, gwid)
    plsc.subcore_barrier()

    def scale(t, b):
        @pl.loop(0, 128, unroll=8)
        def _(k):
            bc = plsc.load_gather(
                vals_all, [jnp.full((16,), t, jnp.int32),
                           jnp.full((16,), k, jnp.int32)])
            bufs[b, k, pl.ds(0, 16)] = bufs[b, k, pl.ds(0, 16)] * bc
            bufs[b, k, pl.ds(16, 16)] = bufs[b, k, pl.ds(16, 16)] * bc

    pltpu.async_copy(h_hbm.at[cols_all.at[0]], bufs.at[0], g0)

    @pl.loop(0, FULL_T, step=2)
    def _(t):
        pltpu.make_async_copy(h_hbm.at[cols_all.at[t]], bufs.at[0], g0).wait()
        d1 = pltpu.async_copy(h_hbm.at[cols_all.at[t + 1]], bufs.at[1], g1)
        scale(t, 0)
        pltpu.sync_copy(bufs.at[0], y_sh.at[rows_all.at[t]], add=True)
        d1.wait()

        @pl.when(t + 2 < FULL_T)
        def _():
            pltpu.async_copy(h_hbm.at[cols_all.at[t + 2]], bufs.at[0], g0)

        scale(t + 1, 1)
        pltpu.sync_copy(bufs.at[1], y_sh.at[rows_all.at[t + 1]], add=True)

    @pl.when(gwid < REM)
    def _():
        pltpu.sync_copy(h_hbm.at[cols_all.at[FULL_T]], bufs.at[0])
        scale(FULL_T, 0)
        pltpu.sync_copy(bufs.at[0], y_sh.at[rows_all.at[FULL_T]], add=True)

    plsc.subcore_barrier()

    @pl.when(sid < 15)
    def _():
        pltpu.sync_copy(y_sh.at[pl.ds(sid * 624, 624)],
                        out_hbm.at[pl.ds(cid * N + sid * 624, 624)])

    @pl.when(sid == 15)
    def _():
        pltpu.sync_copy(y_sh.at[pl.ds(15 * 624, 640)],
                        out_hbm.at[pl.ds(cid * N + 15 * 624, 640)])


def _mm_body(xp_ref, w_ref, h_ref):
    x = xp_ref[0] + xp_ref[1]
    h_ref[...] = jnp.dot(x, w_ref[...], preferred_element_type=jnp.float32)


def _matmul(xp, w):
    bn = 2000
    return pl.pallas_call(
        _mm_body,
        grid=(N // bn,),
        in_specs=[
            pl.BlockSpec((NC, bn, D), lambda i: (0, i, 0)),
            pl.BlockSpec((D, OUT), lambda i: (0, 0)),
        ],
        out_specs=pl.BlockSpec((bn, OUT), lambda i: (i, 0)),
        out_shape=jax.ShapeDtypeStruct((N, OUT), jnp.float32),
    )(xp, w)


def _fin_body(yp_ref, o_ref):
    o_ref[...] = jnp.maximum(yp_ref[0] + yp_ref[1], 0.0)


def _finish(yp):
    bn = 2000
    return pl.pallas_call(
        _fin_body,
        grid=(N // bn,),
        in_specs=[pl.BlockSpec((NC, bn, OUT), lambda i: (0, i, 0))],
        out_specs=pl.BlockSpec((bn, OUT), lambda i: (i, 0)),
        out_shape=jax.ShapeDtypeStruct((N, OUT), jnp.float32),
    )(yp)


def kernel(x_rows, x_cols, x_vals, adj_rows, adj_cols, adj_vals, kernel):
    xf = (x_rows.astype(jnp.int32) * D
          + x_cols.astype(jnp.int32)).reshape(ROWS, 128)
    xv = x_vals.reshape(ROWS, 128)
    ar = adj_rows.astype(jnp.int32).reshape(ROWS, 128)
    ac = adj_cols.astype(jnp.int32).reshape(ROWS, 128)
    av = adj_vals.reshape(ROWS, 128)
    z1 = jnp.zeros((ND,), jnp.float32)
    z2 = jnp.zeros((N, OUT), jnp.float32)

    xd = _stage1(xf, xv, z1)
    h = _matmul(xd.reshape(NC, N, D), kernel)
    yp = _stage2(ar, ac, av, h, z2)
    return _finish(yp.reshape(NC, N, OUT))


# speedup vs baseline: 24.5234x; 1.2728x over previous
import dataclasses
import functools

import jax
import jax.numpy as jnp
from jax import lax
from jax.experimental import pallas as pl
from jax.experimental.pallas import tpu as pltpu
from jax.experimental.pallas import tpu_sc as plsc

N = 10000
D = 128
OUT = 32
NNZ = 320000
ROWS = NNZ // 128
NC = 2
NS = 16
NW = NC * NS
FULL_T = ROWS // NW
REM = ROWS - FULL_T * NW
ND = N * D
SL1 = ND // NS

_mesh = plsc.VectorSubcoreMesh(core_axis_name="c", subcore_axis_name="s")

_cp = pltpu.CompilerParams()
for _f, _v in (("needs_layout_passes", False), ("use_tc_tiling_on_sc", False)):
    if _f in pltpu.CompilerParams.__dataclass_fields__:
        _cp = dataclasses.replace(_cp, **{_f: _v})


def _worker_ids():
    cid = lax.axis_index("c")
    sid = lax.axis_index("s")
    gwid = sid * NC + cid
    cbase = gwid * FULL_T + jnp.minimum(gwid, REM)
    return cid, sid, gwid, cbase


def _bulk_load(pairs, cbase, gwid):
    @pl.when(gwid < REM)
    def _():
        for hbm, vmem in pairs:
            pltpu.sync_copy(hbm.at[pl.ds(cbase, FULL_T + 1)], vmem)

    @pl.when(gwid >= REM)
    def _():
        for hbm, vmem in pairs:
            pltpu.sync_copy(hbm.at[pl.ds(cbase, FULL_T)],
                            vmem.at[pl.ds(0, FULL_T)])


@functools.partial(
    pl.kernel,
    mesh=_mesh,
    compiler_params=_cp,
    out_type=jax.ShapeDtypeStruct((NC * ND,), jnp.float32),
    scratch_types=[
        pltpu.VMEM((FULL_T + 1, 128), jnp.int32),
        pltpu.VMEM((FULL_T + 1, 128), jnp.float32),
        pltpu.VMEM_SHARED((ND,), jnp.float32),
        pltpu.SemaphoreType.DMA,
        pltpu.SemaphoreType.DMA,
    ],
)
def _stage1(xf_hbm, xv_hbm, z_hbm, out_hbm, idx_all, vals_all, xd_sh, s0, s1):
    cid, sid, gwid, cbase = _worker_ids()

    pltpu.sync_copy(z_hbm.at[pl.ds(sid * SL1, SL1)],
                    xd_sh.at[pl.ds(sid * SL1, SL1)])
    _bulk_load([(xf_hbm, idx_all), (xv_hbm, vals_all)], cbase, gwid)
    plsc.subcore_barrier()

    @pl.loop(0, FULL_T, step=2)
    def _(t):
        d0 = pltpu.async_copy(vals_all.at[t], xd_sh.at[idx_all.at[t]],
                              s0, add=True)
        d0.wait()
        d1 = pltpu.async_copy(vals_all.at[t + 1], xd_sh.at[idx_all.at[t + 1]],
                              s1, add=True)
        d1.wait()

    @pl.when(gwid < REM)
    def _():
        pltpu.sync_copy(vals_all.at[FULL_T], xd_sh.at[idx_all.at[FULL_T]],
                        add=True)

    plsc.subcore_barrier()
    pltpu.sync_copy(xd_sh.at[pl.ds(sid * SL1, SL1)],
                    out_hbm.at[pl.ds(cid * ND + sid * SL1, SL1)])


@functools.partial(
    pl.kernel,
    mesh=_mesh,
    compiler_params=_cp,
    out_type=jax.ShapeDtypeStruct((NC * N, OUT), jnp.float32),
    scratch_types=[
        pltpu.VMEM((FULL_T + 1, 128), jnp.int32),
        pltpu.VMEM((FULL_T + 1, 128), jnp.int32),
        pltpu.VMEM((FULL_T + 1, 128), jnp.float32),
        pltpu.VMEM((3, 128, OUT), jnp.float32),
        pltpu.VMEM_SHARED((N, OUT), jnp.float32),
        pltpu.SemaphoreType.DMA,
        pltpu.SemaphoreType.DMA,
        pltpu.SemaphoreType.DMA,
        pltpu.SemaphoreType.DMA,
        pltpu.SemaphoreType.DMA,
        pltpu.SemaphoreType.DMA,
    ],
)
def _stage2(ar_hbm, ac_hbm, av_hbm, h_hbm, z_hbm, out_hbm,
            rows_all, cols_all, vals_all, bufs, y_sh,
            g0, g1, g2, s0, s1, s2):
    cid, sid, gwid, cbase = _worker_ids()

    @pl.when(sid < 15)
    def _():
        pltpu.sync_copy(z_hbm.at[pl.ds(sid * 624, 624)],
                        y_sh.at[pl.ds(sid * 624, 624)])

    @pl.when(sid == 15)
    def _():
        pltpu.sync_copy(z_hbm.at[pl.ds(15 * 624, 640)],
                        y_sh.at[pl.ds(15 * 624, 640)])

    _bulk_load([(ar_hbm, rows_all), (ac_hbm, cols_all), (av_hbm, vals_all)],
               cbase, gwid)
    plsc.subcore_barrier()

    def scale(t, b):
        @pl.loop(0, 8)
        def _(g):
            v16 = vals_all[t, pl.ds(g * 16, 16)]
            for j in range(16):
                k = g * 16 + j
                bc = jnp.full((16,), v16[j], jnp.float32)
                bufs[b, k, pl.ds(0, 16)] = bufs[b, k, pl.ds(0, 16)] * bc
                bufs[b, k, pl.ds(16, 16)] = bufs[b, k, pl.ds(16, 16)] * bc

    nchunks = jnp.where(gwid < REM, FULL_T + 1, FULL_T)
    gsem = (g0, g1, g2)
    ssem = (s0, s1, s2)

    pltpu.async_copy(h_hbm.at[cols_all.at[0]], bufs.at[0], g0)
    pltpu.async_copy(h_hbm.at[cols_all.at[1]], bufs.at[1], g1)

    def chunk_body(c, b):
        bn = (b + 2) % 3
        pltpu.make_async_copy(h_hbm.at[cols_all.at[c]], bufs.at[b],
                              gsem[b]).wait()
        scale(c, b)

        @pl.when(c > 0)
        def _():
            pltpu.make_async_copy(bufs.at[bn], y_sh.at[rows_all.at[c - 1]],
                                  ssem[bn]).wait()

        @pl.when(c + 2 < nchunks)
        def _():
            pltpu.async_copy(h_hbm.at[cols_all.at[c + 2]], bufs.at[bn],
                             gsem[bn])

        pltpu.async_copy(bufs.at[b], y_sh.at[rows_all.at[c]], ssem[b],
                         add=True)

    @pl.loop(0, FULL_T, step=3)
    def _(t):
        chunk_body(t, 0)
        chunk_body(t + 1, 1)
        chunk_body(t + 2, 2)

    pltpu.make_async_copy(bufs.at[2], y_sh.at[rows_all.at[FULL_T - 1]],
                          ssem[2]).wait()

    @pl.when(gwid < REM)
    def _():
        pltpu.make_async_copy(h_hbm.at[cols_all.at[FULL_T]], bufs.at[0],
                              gsem[0]).wait()
        scale(FULL_T, 0)
        pltpu.sync_copy(bufs.at[0], y_sh.at[rows_all.at[FULL_T]], add=True)

    plsc.subcore_barrier()

    @pl.when(sid < 15)
    def _():
        pltpu.sync_copy(y_sh.at[pl.ds(sid * 624, 624)],
                        out_hbm.at[pl.ds(cid * N + sid * 624, 624)])

    @pl.when(sid == 15)
    def _():
        pltpu.sync_copy(y_sh.at[pl.ds(15 * 624, 640)],
                        out_hbm.at[pl.ds(cid * N + 15 * 624, 640)])


def _mm_body(xp_ref, w_ref, h_ref):
    x = xp_ref[0] + xp_ref[1]
    h_ref[...] = jnp.dot(x, w_ref[...], preferred_element_type=jnp.float32)


def _matmul(xp, w):
    bn = 2000
    return pl.pallas_call(
        _mm_body,
        grid=(N // bn,),
        in_specs=[
            pl.BlockSpec((NC, bn, D), lambda i: (0, i, 0)),
            pl.BlockSpec((D, OUT), lambda i: (0, 0)),
        ],
        out_specs=pl.BlockSpec((bn, OUT), lambda i: (i, 0)),
        out_shape=jax.ShapeDtypeStruct((N, OUT), jnp.float32),
    )(xp, w)


def _fin_body(yp_ref, o_ref):
    o_ref[...] = jnp.maximum(yp_ref[0] + yp_ref[1], 0.0)


def _finish(yp):
    bn = 2000
    return pl.pallas_call(
        _fin_body,
        grid=(N // bn,),
        in_specs=[pl.BlockSpec((NC, bn, OUT), lambda i: (0, i, 0))],
        out_specs=pl.BlockSpec((bn, OUT), lambda i: (i, 0)),
        out_shape=jax.ShapeDtypeStruct((N, OUT), jnp.float32),
    )(yp)


def kernel(x_rows, x_cols, x_vals, adj_rows, adj_cols, adj_vals, kernel):
    xf = (x_rows.astype(jnp.int32) * D
          + x_cols.astype(jnp.int32)).reshape(ROWS, 128)
    xv = x_vals.reshape(ROWS, 128)
    ar = adj_rows.astype(jnp.int32).reshape(ROWS, 128)
    ac = adj_cols.astype(jnp.int32).reshape(ROWS, 128)
    av = adj_vals.reshape(ROWS, 128)
    z1 = jnp.zeros((ND,), jnp.float32)
    z2 = jnp.zeros((N, OUT), jnp.float32)

    xd = _stage1(xf, xv, z1)
    h = _matmul(xd.reshape(NC, N, D), kernel)
    yp = _stage2(ar, ac, av, h, z2)
    return _finish(yp.reshape(NC, N, OUT))


# speedup vs baseline: 24.9810x; 1.0187x over previous
import dataclasses
import functools

import jax
import jax.numpy as jnp
from jax import lax
from jax.experimental import pallas as pl
from jax.experimental.pallas import tpu as pltpu
from jax.experimental.pallas import tpu_sc as plsc

N = 10000
D = 128
OUT = 32
NNZ = 320000
ROWS = NNZ // 128
NC = 2
NS = 16
NW = NC * NS
FULL_T = ROWS // NW
REM = ROWS - FULL_T * NW
ND = N * D
SL1 = ND // NS

_mesh = plsc.VectorSubcoreMesh(core_axis_name="c", subcore_axis_name="s")

_cp = pltpu.CompilerParams()
for _f, _v in (("needs_layout_passes", False), ("use_tc_tiling_on_sc", False)):
    if _f in pltpu.CompilerParams.__dataclass_fields__:
        _cp = dataclasses.replace(_cp, **{_f: _v})


def _worker_ids():
    cid = lax.axis_index("c")
    sid = lax.axis_index("s")
    gwid = sid * NC + cid
    cbase = gwid * FULL_T + jnp.minimum(gwid, REM)
    return cid, sid, gwid, cbase


def _bulk_load(pairs, cbase, gwid):
    @pl.when(gwid < REM)
    def _():
        for hbm, vmem in pairs:
            pltpu.sync_copy(hbm.at[pl.ds(cbase, FULL_T + 1)], vmem)

    @pl.when(gwid >= REM)
    def _():
        for hbm, vmem in pairs:
            pltpu.sync_copy(hbm.at[pl.ds(cbase, FULL_T)],
                            vmem.at[pl.ds(0, FULL_T)])


@functools.partial(
    pl.kernel,
    mesh=_mesh,
    compiler_params=_cp,
    out_type=jax.ShapeDtypeStruct((NC * ND,), jnp.float32),
    scratch_types=[
        pltpu.VMEM((FULL_T + 1, 128), jnp.int32),
        pltpu.VMEM((FULL_T + 1, 128), jnp.float32),
        pltpu.VMEM_SHARED((ND,), jnp.float32),
        pltpu.SemaphoreType.DMA,
        pltpu.SemaphoreType.DMA,
    ],
)
def _stage1(xf_hbm, xv_hbm, z_hbm, out_hbm, idx_all, vals_all, xd_sh, s0, s1):
    cid, sid, gwid, cbase = _worker_ids()

    dz = pltpu.async_copy(z_hbm, xd_sh.at[pl.ds(sid * SL1, SL1)], s1)
    _bulk_load([(xf_hbm, idx_all), (xv_hbm, vals_all)], cbase, gwid)
    dz.wait()
    plsc.subcore_barrier()

    @pl.loop(0, FULL_T, step=2)
    def _(t):
        d0 = pltpu.async_copy(vals_all.at[t], xd_sh.at[idx_all.at[t]],
                              s0, add=True)
        d0.wait()
        d1 = pltpu.async_copy(vals_all.at[t + 1], xd_sh.at[idx_all.at[t + 1]],
                              s1, add=True)
        d1.wait()

    @pl.when(gwid < REM)
    def _():
        pltpu.sync_copy(vals_all.at[FULL_T], xd_sh.at[idx_all.at[FULL_T]],
                        add=True)

    plsc.subcore_barrier()
    pltpu.sync_copy(xd_sh.at[pl.ds(sid * SL1, SL1)],
                    out_hbm.at[pl.ds(cid * ND + sid * SL1, SL1)])


@functools.partial(
    pl.kernel,
    mesh=_mesh,
    compiler_params=_cp,
    out_type=jax.ShapeDtypeStruct((NC * N, OUT), jnp.float32),
    scratch_types=[
        pltpu.VMEM((FULL_T + 1, 128), jnp.int32),
        pltpu.VMEM((FULL_T + 1, 128), jnp.int32),
        pltpu.VMEM((FULL_T + 1, 128), jnp.float32),
        pltpu.VMEM((3, 128, OUT), jnp.float32),
        pltpu.VMEM_SHARED((N, OUT), jnp.float32),
        pltpu.SemaphoreType.DMA,
        pltpu.SemaphoreType.DMA,
        pltpu.SemaphoreType.DMA,
        pltpu.SemaphoreType.DMA,
        pltpu.SemaphoreType.DMA,
        pltpu.SemaphoreType.DMA,
    ],
)
def _stage2(ar_hbm, ac_hbm, av_hbm, h_hbm, z_hbm, out_hbm,
            rows_all, cols_all, vals_all, bufs, y_sh,
            g0, g1, g2, s0, s1, s2):
    cid, sid, gwid, cbase = _worker_ids()

    @pl.when(sid < 15)
    def _():
        pltpu.async_copy(z_hbm.at[pl.ds(0, 624)],
                         y_sh.at[pl.ds(sid * 624, 624)], s0)

    @pl.when(sid == 15)
    def _():
        pltpu.async_copy(z_hbm, y_sh.at[pl.ds(15 * 624, 640)], s0)

    _bulk_load([(ar_hbm, rows_all), (ac_hbm, cols_all), (av_hbm, vals_all)],
               cbase, gwid)

    @pl.when(sid < 15)
    def _():
        pltpu.make_async_copy(z_hbm.at[pl.ds(0, 624)],
                              y_sh.at[pl.ds(sid * 624, 624)], s0).wait()

    @pl.when(sid == 15)
    def _():
        pltpu.make_async_copy(z_hbm, y_sh.at[pl.ds(15 * 624, 640)], s0).wait()

    plsc.subcore_barrier()

    def scale(t, b):
        @pl.loop(0, 8)
        def _(g):
            v16 = vals_all[t, pl.ds(g * 16, 16)]
            for j in range(16):
                k = g * 16 + j
                bc = jnp.full((16,), v16[j], jnp.float32)
                bufs[b, k, pl.ds(0, 16)] = bufs[b, k, pl.ds(0, 16)] * bc
                bufs[b, k, pl.ds(16, 16)] = bufs[b, k, pl.ds(16, 16)] * bc

    nchunks = jnp.where(gwid < REM, FULL_T + 1, FULL_T)
    gsem = (g0, g1, g2)
    ssem = (s0, s1, s2)

    pltpu.async_copy(h_hbm.at[cols_all.at[0]], bufs.at[0], g0)
    pltpu.async_copy(h_hbm.at[cols_all.at[1]], bufs.at[1], g1)

    def chunk_body(c, b):
        bn = (b + 2) % 3
        pltpu.make_async_copy(h_hbm.at[cols_all.at[c]], bufs.at[b],
                              gsem[b]).wait()
        scale(c, b)

        @pl.when(c > 0)
        def _():
            pltpu.make_async_copy(bufs.at[bn], y_sh.at[rows_all.at[c - 1]],
                                  ssem[bn]).wait()

        @pl.when(c + 2 < nchunks)
        def _():
            pltpu.async_copy(h_hbm.at[cols_all.at[c + 2]], bufs.at[bn],
                             gsem[bn])

        pltpu.async_copy(bufs.at[b], y_sh.at[rows_all.at[c]], ssem[b],
                         add=True)

    @pl.loop(0, FULL_T, step=3)
    def _(t):
        chunk_body(t, 0)
        chunk_body(t + 1, 1)
        chunk_body(t + 2, 2)

    pltpu.make_async_copy(bufs.at[2], y_sh.at[rows_all.at[FULL_T - 1]],
                          ssem[2]).wait()

    @pl.when(gwid < REM)
    def _():
        pltpu.make_async_copy(h_hbm.at[cols_all.at[FULL_T]], bufs.at[0],
                              gsem[0]).wait()
        scale(FULL_T, 0)
        pltpu.sync_copy(bufs.at[0], y_sh.at[rows_all.at[FULL_T]], add=True)

    plsc.subcore_barrier()

    @pl.when(sid < 15)
    def _():
        pltpu.sync_copy(y_sh.at[pl.ds(sid * 624, 624)],
                        out_hbm.at[pl.ds(cid * N + sid * 624, 624)])

    @pl.when(sid == 15)
    def _():
        pltpu.sync_copy(y_sh.at[pl.ds(15 * 624, 640)],
                        out_hbm.at[pl.ds(cid * N + 15 * 624, 640)])


def _mm_body(xp_ref, w_ref, h_ref):
    x = xp_ref[0] + xp_ref[1]
    h_ref[...] = jnp.dot(x, w_ref[...], preferred_element_type=jnp.float32)


def _matmul(xp, w):
    bn = 2000
    return pl.pallas_call(
        _mm_body,
        grid=(N // bn,),
        in_specs=[
            pl.BlockSpec((NC, bn, D), lambda i: (0, i, 0)),
            pl.BlockSpec((D, OUT), lambda i: (0, 0)),
        ],
        out_specs=pl.BlockSpec((bn, OUT), lambda i: (i, 0)),
        out_shape=jax.ShapeDtypeStruct((N, OUT), jnp.float32),
    )(xp, w)


def _fin_body(yp_ref, o_ref):
    o_ref[...] = jnp.maximum(yp_ref[0] + yp_ref[1], 0.0)


def _finish(yp):
    bn = 2000
    return pl.pallas_call(
        _fin_body,
        grid=(N // bn,),
        in_specs=[pl.BlockSpec((NC, bn, OUT), lambda i: (0, i, 0))],
        out_specs=pl.BlockSpec((bn, OUT), lambda i: (i, 0)),
        out_shape=jax.ShapeDtypeStruct((N, OUT), jnp.float32),
    )(yp)


def kernel(x_rows, x_cols, x_vals, adj_rows, adj_cols, adj_vals, kernel):
    xf = (x_rows.astype(jnp.int32) * D
          + x_cols.astype(jnp.int32)).reshape(ROWS, 128)
    xv = x_vals.reshape(ROWS, 128)
    ar = adj_rows.astype(jnp.int32).reshape(ROWS, 128)
    ac = adj_cols.astype(jnp.int32).reshape(ROWS, 128)
    av = adj_vals.reshape(ROWS, 128)
    z1 = jnp.zeros((SL1,), jnp.float32)
    z2 = jnp.zeros((640, OUT), jnp.float32)

    xd = _stage1(xf, xv, z1)
    h = _matmul(xd.reshape(NC, N, D), kernel)
    yp = _stage2(ar, ac, av, h, z2)
    return _finish(yp.reshape(NC, N, OUT))


# speedup vs baseline: 25.4692x; 1.0195x over previous
import dataclasses
import functools

import jax
import jax.numpy as jnp
from jax import lax
from jax.experimental import pallas as pl
from jax.experimental.pallas import tpu as pltpu
from jax.experimental.pallas import tpu_sc as plsc

N = 10000
D = 128
OUT = 32
NNZ = 320000
ROWS = NNZ // 128
NC = 2
NS = 16
NW = NC * NS
FULL_T = ROWS // NW
REM = ROWS - FULL_T * NW
ND = N * D
SL1 = ND // NS

_mesh = plsc.VectorSubcoreMesh(core_axis_name="c", subcore_axis_name="s")

_cp = pltpu.CompilerParams()
for _f, _v in (("needs_layout_passes", False), ("use_tc_tiling_on_sc", False)):
    if _f in pltpu.CompilerParams.__dataclass_fields__:
        _cp = dataclasses.replace(_cp, **{_f: _v})


W1 = 256
ROWS1 = NNZ // W1
FT1 = ROWS1 // NW
REM1 = ROWS1 - FT1 * NW


def _worker_ids():
    cid = lax.axis_index("c")
    sid = lax.axis_index("s")
    gwid = sid * NC + cid
    return cid, sid, gwid


def _bulk_load(pairs, cbase, gwid, ft, rem):
    @pl.when(gwid < rem)
    def _():
        for hbm, vmem in pairs:
            pltpu.sync_copy(hbm.at[pl.ds(cbase, ft + 1)], vmem)

    @pl.when(gwid >= rem)
    def _():
        for hbm, vmem in pairs:
            pltpu.sync_copy(hbm.at[pl.ds(cbase, ft)],
                            vmem.at[pl.ds(0, ft)])


@functools.partial(
    pl.kernel,
    mesh=_mesh,
    compiler_params=_cp,
    out_type=jax.ShapeDtypeStruct((NC * ND,), jnp.float32),
    scratch_types=[
        pltpu.VMEM((FT1 + 1, W1), jnp.int32),
        pltpu.VMEM((FT1 + 1, W1), jnp.float32),
        pltpu.VMEM_SHARED((ND,), jnp.float32),
        pltpu.SemaphoreType.DMA,
        pltpu.SemaphoreType.DMA,
    ],
)
def _stage1(xf_hbm, xv_hbm, z_hbm, out_hbm, idx_all, vals_all, xd_sh, s0, s1):
    cid, sid, gwid = _worker_ids()
    cbase = gwid * FT1 + jnp.minimum(gwid, REM1)

    dz = pltpu.async_copy(z_hbm, xd_sh.at[pl.ds(sid * SL1, SL1)], s1)
    _bulk_load([(xf_hbm, idx_all), (xv_hbm, vals_all)], cbase, gwid, FT1, REM1)
    dz.wait()
    plsc.subcore_barrier()

    @pl.loop(0, FT1)
    def _(t):
        pltpu.sync_copy(vals_all.at[t], xd_sh.at[idx_all.at[t]], add=True)

    @pl.when(gwid < REM1)
    def _():
        pltpu.sync_copy(vals_all.at[FT1], xd_sh.at[idx_all.at[FT1]],
                        add=True)

    plsc.subcore_barrier()
    pltpu.sync_copy(xd_sh.at[pl.ds(sid * SL1, SL1)],
                    out_hbm.at[pl.ds(cid * ND + sid * SL1, SL1)])


@functools.partial(
    pl.kernel,
    mesh=_mesh,
    compiler_params=_cp,
    out_type=jax.ShapeDtypeStruct((NC * N, OUT), jnp.float32),
    scratch_types=[
        pltpu.VMEM((FULL_T + 1, 128), jnp.int32),
        pltpu.VMEM((FULL_T + 1, 128), jnp.int32),
        pltpu.VMEM((FULL_T + 1, 128), jnp.float32),
        pltpu.VMEM((3, 128, OUT), jnp.float32),
        pltpu.VMEM_SHARED((N, OUT), jnp.float32),
        pltpu.SemaphoreType.DMA,
        pltpu.SemaphoreType.DMA,
        pltpu.SemaphoreType.DMA,
        pltpu.SemaphoreType.DMA,
        pltpu.SemaphoreType.DMA,
        pltpu.SemaphoreType.DMA,
    ],
)
def _stage2(ar_hbm, ac_hbm, av_hbm, h_hbm, z_hbm, out_hbm,
            rows_all, cols_all, vals_all, bufs, y_sh,
            g0, g1, g2, s0, s1, s2):
    cid, sid, gwid = _worker_ids()
    cbase = gwid * FULL_T + jnp.minimum(gwid, REM)

    @pl.when(sid < 15)
    def _():
        pltpu.async_copy(z_hbm.at[pl.ds(0, 624)],
                         y_sh.at[pl.ds(sid * 624, 624)], s0)

    @pl.when(sid == 15)
    def _():
        pltpu.async_copy(z_hbm, y_sh.at[pl.ds(15 * 624, 640)], s0)

    _bulk_load([(ar_hbm, rows_all), (ac_hbm, cols_all), (av_hbm, vals_all)],
               cbase, gwid, FULL_T, REM)

    @pl.when(sid < 15)
    def _():
        pltpu.make_async_copy(z_hbm.at[pl.ds(0, 624)],
                              y_sh.at[pl.ds(sid * 624, 624)], s0).wait()

    @pl.when(sid == 15)
    def _():
        pltpu.make_async_copy(z_hbm, y_sh.at[pl.ds(15 * 624, 640)], s0).wait()

    plsc.subcore_barrier()

    def scale(t, b):
        @pl.loop(0, 8)
        def _(g):
            v16 = vals_all[t, pl.ds(g * 16, 16)]
            for j in range(16):
                k = g * 16 + j
                bc = jnp.full((16,), v16[j], jnp.float32)
                bufs[b, k, pl.ds(0, 16)] = bufs[b, k, pl.ds(0, 16)] * bc
                bufs[b, k, pl.ds(16, 16)] = bufs[b, k, pl.ds(16, 16)] * bc

    nchunks = jnp.where(gwid < REM, FULL_T + 1, FULL_T)
    gsem = (g0, g1, g2)
    ssem = (s0, s1, s2)

    pltpu.async_copy(h_hbm.at[cols_all.at[0]], bufs.at[0], g0)
    pltpu.async_copy(h_hbm.at[cols_all.at[1]], bufs.at[1], g1)

    def chunk_body(c, b):
        bn = (b + 2) % 3
        pltpu.make_async_copy(h_hbm.at[cols_all.at[c]], bufs.at[b],
                              gsem[b]).wait()
        scale(c, b)

        @pl.when(c > 0)
        def _():
            pltpu.make_async_copy(bufs.at[bn], y_sh.at[rows_all.at[c - 1]],
                                  ssem[bn]).wait()

        @pl.when(c + 2 < nchunks)
        def _():
            pltpu.async_copy(h_hbm.at[cols_all.at[c + 2]], bufs.at[bn],
                             gsem[bn])

        pltpu.async_copy(bufs.at[b], y_sh.at[rows_all.at[c]], ssem[b],
                         add=True)

    @pl.loop(0, FULL_T, step=3)
    def _(t):
        chunk_body(t, 0)
        chunk_body(t + 1, 1)
        chunk_body(t + 2, 2)

    pltpu.make_async_copy(bufs.at[2], y_sh.at[rows_all.at[FULL_T - 1]],
                          ssem[2]).wait()

    @pl.when(gwid < REM)
    def _():
        pltpu.make_async_copy(h_hbm.at[cols_all.at[FULL_T]], bufs.at[0],
                              gsem[0]).wait()
        scale(FULL_T, 0)
        pltpu.sync_copy(bufs.at[0], y_sh.at[rows_all.at[FULL_T]], add=True)

    plsc.subcore_barrier()

    @pl.when(sid < 15)
    def _():
        pltpu.sync_copy(y_sh.at[pl.ds(sid * 624, 624)],
                        out_hbm.at[pl.ds(cid * N + sid * 624, 624)])

    @pl.when(sid == 15)
    def _():
        pltpu.sync_copy(y_sh.at[pl.ds(15 * 624, 640)],
                        out_hbm.at[pl.ds(cid * N + 15 * 624, 640)])


def _mm_body(xp_ref, w_ref, h_ref):
    x = xp_ref[0] + xp_ref[1]
    h_ref[...] = jnp.dot(x, w_ref[...], preferred_element_type=jnp.float32)


def _matmul(xp, w):
    bn = 2000
    return pl.pallas_call(
        _mm_body,
        grid=(N // bn,),
        in_specs=[
            pl.BlockSpec((NC, bn, D), lambda i: (0, i, 0)),
            pl.BlockSpec((D, OUT), lambda i: (0, 0)),
        ],
        out_specs=pl.BlockSpec((bn, OUT), lambda i: (i, 0)),
        out_shape=jax.ShapeDtypeStruct((N, OUT), jnp.float32),
    )(xp, w)


def _fin_body(yp_ref, o_ref):
    o_ref[...] = jnp.maximum(yp_ref[0] + yp_ref[1], 0.0)


def _finish(yp):
    bn = 2000
    return pl.pallas_call(
        _fin_body,
        grid=(N // bn,),
        in_specs=[pl.BlockSpec((NC, bn, OUT), lambda i: (0, i, 0))],
        out_specs=pl.BlockSpec((bn, OUT), lambda i: (i, 0)),
        out_shape=jax.ShapeDtypeStruct((N, OUT), jnp.float32),
    )(yp)


def kernel(x_rows, x_cols, x_vals, adj_rows, adj_cols, adj_vals, kernel):
    xf = (x_rows.astype(jnp.int32) * D
          + x_cols.astype(jnp.int32)).reshape(ROWS1, W1)
    xv = x_vals.reshape(ROWS1, W1)
    ar = adj_rows.astype(jnp.int32).reshape(ROWS, 128)
    ac = adj_cols.astype(jnp.int32).reshape(ROWS, 128)
    av = adj_vals.reshape(ROWS, 128)
    z1 = jnp.zeros((SL1,), jnp.float32)
    z2 = jnp.zeros((640, OUT), jnp.float32)

    xd = _stage1(xf, xv, z1)
    h = _matmul(xd.reshape(NC, N, D), kernel)
    yp = _stage2(ar, ac, av, h, z2)
    return _finish(yp.reshape(NC, N, OUT))


# speedup vs baseline: 25.8058x; 1.0132x over previous
import dataclasses
import functools

import jax
import jax.numpy as jnp
from jax import lax
from jax.experimental import pallas as pl
from jax.experimental.pallas import tpu as pltpu
from jax.experimental.pallas import tpu_sc as plsc

N = 10000
D = 128
OUT = 32
NNZ = 320000
ROWS = NNZ // 128
NC = 2
NS = 16
NW = NC * NS
FULL_T = ROWS // NW
REM = ROWS - FULL_T * NW
ND = N * D
SL1 = ND // NS

_mesh = plsc.VectorSubcoreMesh(core_axis_name="c", subcore_axis_name="s")

_cp = pltpu.CompilerParams()
for _f, _v in (("needs_layout_passes", False), ("use_tc_tiling_on_sc", False)):
    if _f in pltpu.CompilerParams.__dataclass_fields__:
        _cp = dataclasses.replace(_cp, **{_f: _v})


W1 = 512
ROWS1 = NNZ // W1
FT1 = ROWS1 // NW
REM1 = ROWS1 - FT1 * NW


def _worker_ids():
    cid = lax.axis_index("c")
    sid = lax.axis_index("s")
    gwid = sid * NC + cid
    return cid, sid, gwid


def _bulk_load(pairs, cbase, gwid, ft, rem):
    @pl.when(gwid < rem)
    def _():
        for hbm, vmem in pairs:
            pltpu.sync_copy(hbm.at[pl.ds(cbase, ft + 1)], vmem)

    @pl.when(gwid >= rem)
    def _():
        for hbm, vmem in pairs:
            pltpu.sync_copy(hbm.at[pl.ds(cbase, ft)],
                            vmem.at[pl.ds(0, ft)])


@functools.partial(
    pl.kernel,
    mesh=_mesh,
    compiler_params=_cp,
    out_type=jax.ShapeDtypeStruct((NC * ND,), jnp.float32),
    scratch_types=[
        pltpu.VMEM((FT1 + 1, W1), jnp.int32),
        pltpu.VMEM((FT1 + 1, W1), jnp.float32),
        pltpu.VMEM_SHARED((ND,), jnp.float32),
        pltpu.SemaphoreType.DMA,
        pltpu.SemaphoreType.DMA,
    ],
)
def _stage1(xf_hbm, xv_hbm, z_hbm, out_hbm, idx_all, vals_all, xd_sh, s0, s1):
    cid, sid, gwid = _worker_ids()
    cbase = gwid * FT1 + jnp.minimum(gwid, REM1)

    dz = pltpu.async_copy(z_hbm, xd_sh.at[pl.ds(sid * SL1, SL1)], s1)
    _bulk_load([(xf_hbm, idx_all), (xv_hbm, vals_all)], cbase, gwid, FT1, REM1)
    dz.wait()
    plsc.subcore_barrier()

    @pl.loop(0, FT1)
    def _(t):
        pltpu.sync_copy(vals_all.at[t], xd_sh.at[idx_all.at[t]], add=True)

    @pl.when(gwid < REM1)
    def _():
        pltpu.sync_copy(vals_all.at[FT1], xd_sh.at[idx_all.at[FT1]],
                        add=True)

    plsc.subcore_barrier()
    pltpu.sync_copy(xd_sh.at[pl.ds(sid * SL1, SL1)],
                    out_hbm.at[pl.ds(cid * ND + sid * SL1, SL1)])


@functools.partial(
    pl.kernel,
    mesh=_mesh,
    compiler_params=_cp,
    out_type=jax.ShapeDtypeStruct((NC * N, OUT), jnp.float32),
    scratch_types=[
        pltpu.VMEM((FULL_T + 1, 128), jnp.int32),
        pltpu.VMEM((FULL_T + 1, 128), jnp.int32),
        pltpu.VMEM((FULL_T + 1, 128), jnp.float32),
        pltpu.VMEM((3, 128, OUT), jnp.float32),
        pltpu.VMEM_SHARED((N, OUT), jnp.float32),
        pltpu.SemaphoreType.DMA,
        pltpu.SemaphoreType.DMA,
        pltpu.SemaphoreType.DMA,
        pltpu.SemaphoreType.DMA,
        pltpu.SemaphoreType.DMA,
        pltpu.SemaphoreType.DMA,
    ],
)
def _stage2(ar_hbm, ac_hbm, av_hbm, h_hbm, z_hbm, out_hbm,
            rows_all, cols_all, vals_all, bufs, y_sh,
            g0, g1, g2, s0, s1, s2):
    cid, sid, gwid = _worker_ids()
    cbase = gwid * FULL_T + jnp.minimum(gwid, REM)

    @pl.when(sid < 15)
    def _():
        pltpu.async_copy(z_hbm.at[pl.ds(0, 624)],
                         y_sh.at[pl.ds(sid * 624, 624)], s0)

    @pl.when(sid == 15)
    def _():
        pltpu.async_copy(z_hbm, y_sh.at[pl.ds(15 * 624, 640)], s0)

    _bulk_load([(ar_hbm, rows_all), (ac_hbm, cols_all), (av_hbm, vals_all)],
               cbase, gwid, FULL_T, REM)

    @pl.when(sid < 15)
    def _():
        pltpu.make_async_copy(z_hbm.at[pl.ds(0, 624)],
                              y_sh.at[pl.ds(sid * 624, 624)], s0).wait()

    @pl.when(sid == 15)
    def _():
        pltpu.make_async_copy(z_hbm, y_sh.at[pl.ds(15 * 624, 640)], s0).wait()

    plsc.subcore_barrier()

    def scale(t, b):
        @pl.loop(0, 8)
        def _(g):
            v16 = vals_all[t, pl.ds(g * 16, 16)]
            for j in range(16):
                k = g * 16 + j
                bc = jnp.full((16,), v16[j], jnp.float32)
                bufs[b, k, pl.ds(0, 16)] = bufs[b, k, pl.ds(0, 16)] * bc
                bufs[b, k, pl.ds(16, 16)] = bufs[b, k, pl.ds(16, 16)] * bc

    nchunks = jnp.where(gwid < REM, FULL_T + 1, FULL_T)
    gsem = (g0, g1, g2)
    ssem = (s0, s1, s2)

    pltpu.async_copy(h_hbm.at[cols_all.at[0]], bufs.at[0], g0)
    pltpu.async_copy(h_hbm.at[cols_all.at[1]], bufs.at[1], g1)

    def chunk_body(c, b):
        bn = (b + 2) % 3
        pltpu.make_async_copy(h_hbm.at[cols_all.at[c]], bufs.at[b],
                              gsem[b]).wait()
        scale(c, b)

        @pl.when(c > 0)
        def _():
            pltpu.make_async_copy(bufs.at[bn], y_sh.at[rows_all.at[c - 1]],
                                  ssem[bn]).wait()

        @pl.when(c + 2 < nchunks)
        def _():
            pltpu.async_copy(h_hbm.at[cols_all.at[c + 2]], bufs.at[bn],
                             gsem[bn])

        pltpu.async_copy(bufs.at[b], y_sh.at[rows_all.at[c]], ssem[b],
                         add=True)

    @pl.loop(0, FULL_T, step=3)
    def _(t):
        chunk_body(t, 0)
        chunk_body(t + 1, 1)
        chunk_body(t + 2, 2)

    pltpu.make_async_copy(bufs.at[2], y_sh.at[rows_all.at[FULL_T - 1]],
                          ssem[2]).wait()

    @pl.when(gwid < REM)
    def _():
        pltpu.make_async_copy(h_hbm.at[cols_all.at[FULL_T]], bufs.at[0],
                              gsem[0]).wait()
        scale(FULL_T, 0)
        pltpu.sync_copy(bufs.at[0], y_sh.at[rows_all.at[FULL_T]], add=True)

    plsc.subcore_barrier()

    @pl.when(sid < 15)
    def _():
        pltpu.sync_copy(y_sh.at[pl.ds(sid * 624, 624)],
                        out_hbm.at[pl.ds(cid * N + sid * 624, 624)])

    @pl.when(sid == 15)
    def _():
        pltpu.sync_copy(y_sh.at[pl.ds(15 * 624, 640)],
                        out_hbm.at[pl.ds(cid * N + 15 * 624, 640)])


def _mm_body(xp_ref, w_ref, h_ref):
    x = xp_ref[0] + xp_ref[1]
    h_ref[...] = jnp.dot(x, w_ref[...], preferred_element_type=jnp.float32)


def _matmul(xp, w):
    bn = 2000
    return pl.pallas_call(
        _mm_body,
        grid=(N // bn,),
        in_specs=[
            pl.BlockSpec((NC, bn, D), lambda i: (0, i, 0)),
            pl.BlockSpec((D, OUT), lambda i: (0, 0)),
        ],
        out_specs=pl.BlockSpec((bn, OUT), lambda i: (i, 0)),
        out_shape=jax.ShapeDtypeStruct((N, OUT), jnp.float32),
    )(xp, w)


def _fin_body(yp_ref, o_ref):
    o_ref[...] = jnp.maximum(yp_ref[0] + yp_ref[1], 0.0)


def _finish(yp):
    bn = 2000
    return pl.pallas_call(
        _fin_body,
        grid=(N // bn,),
        in_specs=[pl.BlockSpec((NC, bn, OUT), lambda i: (0, i, 0))],
        out_specs=pl.BlockSpec((bn, OUT), lambda i: (i, 0)),
        out_shape=jax.ShapeDtypeStruct((N, OUT), jnp.float32),
    )(yp)


def kernel(x_rows, x_cols, x_vals, adj_rows, adj_cols, adj_vals, kernel):
    xf = (x_rows.astype(jnp.int32) * D
          + x_cols.astype(jnp.int32)).reshape(ROWS1, W1)
    xv = x_vals.reshape(ROWS1, W1)
    ar = adj_rows.astype(jnp.int32).reshape(ROWS, 128)
    ac = adj_cols.astype(jnp.int32).reshape(ROWS, 128)
    av = adj_vals.reshape(ROWS, 128)
    z1 = jnp.zeros((SL1,), jnp.float32)
    z2 = jnp.zeros((640, OUT), jnp.float32)

    xd = _stage1(xf, xv, z1)
    h = _matmul(xd.reshape(NC, N, D), kernel)
    yp = _stage2(ar, ac, av, h, z2)
    return _finish(yp.reshape(NC, N, OUT))


# speedup vs baseline: 25.8223x; 1.0006x over previous
import dataclasses
import functools

import jax
import jax.numpy as jnp
from jax import lax
from jax.experimental import pallas as pl
from jax.experimental.pallas import tpu as pltpu
from jax.experimental.pallas import tpu_sc as plsc

N = 10000
D = 128
OUT = 32
NNZ = 320000
ROWS = NNZ // 128
NC = 2
NS = 16
NW = NC * NS
FULL_T = ROWS // NW
REM = ROWS - FULL_T * NW
ND = N * D
SL1 = ND // NS

_mesh = plsc.VectorSubcoreMesh(core_axis_name="c", subcore_axis_name="s")

_cp = pltpu.CompilerParams()
for _f, _v in (("needs_layout_passes", False), ("use_tc_tiling_on_sc", False)):
    if _f in pltpu.CompilerParams.__dataclass_fields__:
        _cp = dataclasses.replace(_cp, **{_f: _v})


W1 = 512
ROWS1 = NNZ // W1
FT1 = ROWS1 // NW
REM1 = ROWS1 - FT1 * NW


def _worker_ids():
    cid = lax.axis_index("c")
    sid = lax.axis_index("s")
    gwid = sid * NC + cid
    return cid, sid, gwid


def _bulk_load(pairs, cbase, gwid, ft, rem):
    @pl.when(gwid < rem)
    def _():
        for hbm, vmem in pairs:
            pltpu.sync_copy(hbm.at[pl.ds(cbase, ft + 1)], vmem)

    @pl.when(gwid >= rem)
    def _():
        for hbm, vmem in pairs:
            pltpu.sync_copy(hbm.at[pl.ds(cbase, ft)],
                            vmem.at[pl.ds(0, ft)])


@functools.partial(
    pl.kernel,
    mesh=_mesh,
    compiler_params=_cp,
    out_type=jax.ShapeDtypeStruct((NC * ND,), jnp.float32),
    scratch_types=[
        pltpu.VMEM((FT1 + 1, W1), jnp.int32),
        pltpu.VMEM((FT1 + 1, W1), jnp.float32),
        pltpu.VMEM_SHARED((ND,), jnp.float32),
        pltpu.SemaphoreType.DMA,
        pltpu.SemaphoreType.DMA,
    ],
)
def _stage1(xf_hbm, xv_hbm, z_hbm, out_hbm, idx_all, vals_all, xd_sh, s0, s1):
    cid, sid, gwid = _worker_ids()
    cbase = gwid * FT1 + jnp.minimum(gwid, REM1)

    dz = pltpu.async_copy(z_hbm, xd_sh.at[pl.ds(sid * SL1, SL1)], s1)
    _bulk_load([(xf_hbm, idx_all), (xv_hbm, vals_all)], cbase, gwid, FT1, REM1)
    dz.wait()
    plsc.subcore_barrier()

    @pl.loop(0, FT1)
    def _(t):
        pltpu.sync_copy(vals_all.at[t], xd_sh.at[idx_all.at[t]], add=True)

    @pl.when(gwid < REM1)
    def _():
        pltpu.sync_copy(vals_all.at[FT1], xd_sh.at[idx_all.at[FT1]],
                        add=True)

    plsc.subcore_barrier()
    pltpu.sync_copy(xd_sh.at[pl.ds(sid * SL1, SL1)],
                    out_hbm.at[pl.ds(cid * ND + sid * SL1, SL1)])


@functools.partial(
    pl.kernel,
    mesh=_mesh,
    compiler_params=_cp,
    out_type=jax.ShapeDtypeStruct((NC * N, OUT), jnp.float32),
    scratch_types=[
        pltpu.VMEM((FULL_T + 1, 128), jnp.int32),
        pltpu.VMEM((FULL_T + 1, 128), jnp.int32),
        pltpu.VMEM((FULL_T + 1, 128), jnp.float32),
        pltpu.VMEM((3, 128, OUT), jnp.float32),
        pltpu.VMEM_SHARED((N, OUT), jnp.float32),
        pltpu.SemaphoreType.DMA,
        pltpu.SemaphoreType.DMA,
        pltpu.SemaphoreType.DMA,
        pltpu.SemaphoreType.DMA,
        pltpu.SemaphoreType.DMA,
        pltpu.SemaphoreType.DMA,
    ],
)
def _stage2(ar_hbm, ac_hbm, av_hbm, h_hbm, z_hbm, out_hbm,
            rows_all, cols_all, vals_all, bufs, y_sh,
            g0, g1, g2, s0, s1, s2):
    cid, sid, gwid = _worker_ids()
    cbase = gwid * FULL_T + jnp.minimum(gwid, REM)

    @pl.when(sid < 15)
    def _():
        pltpu.async_copy(z_hbm.at[pl.ds(0, 624)],
                         y_sh.at[pl.ds(sid * 624, 624)], s0)

    @pl.when(sid == 15)
    def _():
        pltpu.async_copy(z_hbm, y_sh.at[pl.ds(15 * 624, 640)], s0)

    _bulk_load([(ar_hbm, rows_all), (ac_hbm, cols_all), (av_hbm, vals_all)],
               cbase, gwid, FULL_T, REM)

    @pl.when(sid < 15)
    def _():
        pltpu.make_async_copy(z_hbm.at[pl.ds(0, 624)],
                              y_sh.at[pl.ds(sid * 624, 624)], s0).wait()

    @pl.when(sid == 15)
    def _():
        pltpu.make_async_copy(z_hbm, y_sh.at[pl.ds(15 * 624, 640)], s0).wait()

    plsc.subcore_barrier()

    def scale(t, b):
        @pl.loop(0, 8)
        def _(g):
            v16 = vals_all[t, pl.ds(g * 16, 16)]
            for j in range(16):
                k = g * 16 + j
                bc = jnp.full((16,), v16[j], jnp.float32)
                bufs[b, k, pl.ds(0, 16)] = bufs[b, k, pl.ds(0, 16)] * bc
                bufs[b, k, pl.ds(16, 16)] = bufs[b, k, pl.ds(16, 16)] * bc

    nchunks = jnp.where(gwid < REM, FULL_T + 1, FULL_T)
    gsem = (g0, g1, g2)
    ssem = (s0, s1, s2)

    pltpu.async_copy(h_hbm.at[cols_all.at[0]], bufs.at[0], g0)
    pltpu.async_copy(h_hbm.at[cols_all.at[1]], bufs.at[1], g1)

    def chunk_body(c, b):
        bn = (b + 2) % 3
        pltpu.make_async_copy(h_hbm.at[cols_all.at[c]], bufs.at[b],
                              gsem[b]).wait()
        scale(c, b)

        @pl.when(c > 0)
        def _():
            pltpu.make_async_copy(bufs.at[bn], y_sh.at[rows_all.at[c - 1]],
                                  ssem[bn]).wait()

        @pl.when(c + 2 < nchunks)
        def _():
            pltpu.async_copy(h_hbm.at[cols_all.at[c + 2]], bufs.at[bn],
                             gsem[bn])

        pltpu.async_copy(bufs.at[b], y_sh.at[rows_all.at[c]], ssem[b],
                         add=True)

    @pl.loop(0, FULL_T, step=3)
    def _(t):
        chunk_body(t, 0)
        chunk_body(t + 1, 1)
        chunk_body(t + 2, 2)

    pltpu.make_async_copy(bufs.at[2], y_sh.at[rows_all.at[FULL_T - 1]],
                          ssem[2]).wait()

    @pl.when(gwid < REM)
    def _():
        pltpu.make_async_copy(h_hbm.at[cols_all.at[FULL_T]], bufs.at[0],
                              gsem[0]).wait()
        scale(FULL_T, 0)
        pltpu.sync_copy(bufs.at[0], y_sh.at[rows_all.at[FULL_T]], add=True)

    plsc.subcore_barrier()

    @pl.when(sid < 15)
    def _():
        pltpu.sync_copy(y_sh.at[pl.ds(sid * 624, 624)],
                        out_hbm.at[pl.ds(cid * N + sid * 624, 624)])

    @pl.when(sid == 15)
    def _():
        pltpu.sync_copy(y_sh.at[pl.ds(15 * 624, 640)],
                        out_hbm.at[pl.ds(cid * N + 15 * 624, 640)])


def _mm_body(x0_ref, x1_ref, w_ref, h_ref):
    bn = h_ref.shape[0]
    x = (x0_ref[...] + x1_ref[...]).reshape(bn, D)
    h_ref[...] = jnp.dot(x, w_ref[...], preferred_element_type=jnp.float32)


def _matmul(xp, w):
    bn = 2000
    nb = N // bn
    return pl.pallas_call(
        _mm_body,
        grid=(nb,),
        in_specs=[
            pl.BlockSpec((bn * D,), lambda i: (i,)),
            pl.BlockSpec((bn * D,), lambda i: (i + nb,)),
            pl.BlockSpec((D, OUT), lambda i: (0, 0)),
        ],
        out_specs=pl.BlockSpec((bn, OUT), lambda i: (i, 0)),
        out_shape=jax.ShapeDtypeStruct((N, OUT), jnp.float32),
    )(xp, xp, w)


def _fin_body(y0_ref, y1_ref, o_ref):
    o_ref[...] = jnp.maximum(y0_ref[...] + y1_ref[...], 0.0)


def _finish(yp):
    bn = 2000
    nb = N // bn
    return pl.pallas_call(
        _fin_body,
        grid=(nb,),
        in_specs=[
            pl.BlockSpec((bn, OUT), lambda i: (i, 0)),
            pl.BlockSpec((bn, OUT), lambda i: (i + nb, 0)),
        ],
        out_specs=pl.BlockSpec((bn, OUT), lambda i: (i, 0)),
        out_shape=jax.ShapeDtypeStruct((N, OUT), jnp.float32),
    )(yp, yp)


def kernel(x_rows, x_cols, x_vals, adj_rows, adj_cols, adj_vals, kernel):
    xf = (x_rows.astype(jnp.int32) * D
          + x_cols.astype(jnp.int32)).reshape(ROWS1, W1)
    xv = x_vals.reshape(ROWS1, W1)
    ar = adj_rows.astype(jnp.int32).reshape(ROWS, 128)
    ac = adj_cols.astype(jnp.int32).reshape(ROWS, 128)
    av = adj_vals.reshape(ROWS, 128)
    z1 = jnp.zeros((SL1,), jnp.float32)
    z2 = jnp.zeros((640, OUT), jnp.float32)

    xd = _stage1(xf, xv, z1)
    h = _matmul(xd, kernel)
    yp = _stage2(ar, ac, av, h, z2)
    return _finish(yp)


# speedup vs baseline: 26.0348x; 1.0082x over previous
import dataclasses
import functools

import jax
import jax.numpy as jnp
from jax import lax
from jax.experimental import pallas as pl
from jax.experimental.pallas import tpu as pltpu
from jax.experimental.pallas import tpu_sc as plsc

N = 10000
D = 128
OUT = 32
NNZ = 320000
ROWS = NNZ // 128
NC = 2
NS = 16
NW = NC * NS
FULL_T = ROWS // NW
REM = ROWS - FULL_T * NW
ND = N * D
SL1 = ND // NS

_mesh = plsc.VectorSubcoreMesh(core_axis_name="c", subcore_axis_name="s")

_cp = pltpu.CompilerParams()
for _f, _v in (("needs_layout_passes", False), ("use_tc_tiling_on_sc", False)):
    if _f in pltpu.CompilerParams.__dataclass_fields__:
        _cp = dataclasses.replace(_cp, **{_f: _v})


W1 = 512
ROWS1 = NNZ // W1
FT1 = ROWS1 // NW
REM1 = ROWS1 - FT1 * NW


def _worker_ids():
    cid = lax.axis_index("c")
    sid = lax.axis_index("s")
    gwid = sid * NC + cid
    return cid, sid, gwid


def _bulk_load(pairs, cbase, gwid, ft, rem, sems):
    @pl.when(gwid < rem)
    def _():
        ds = [pltpu.async_copy(hbm.at[pl.ds(cbase, ft + 1)], vmem, sem)
              for (hbm, vmem), sem in zip(pairs, sems)]
        for d in ds:
            d.wait()

    @pl.when(gwid >= rem)
    def _():
        ds = [pltpu.async_copy(hbm.at[pl.ds(cbase, ft)],
                               vmem.at[pl.ds(0, ft)], sem)
              for (hbm, vmem), sem in zip(pairs, sems)]
        for d in ds:
            d.wait()


@functools.partial(
    pl.kernel,
    mesh=_mesh,
    compiler_params=_cp,
    out_type=jax.ShapeDtypeStruct((NC * ND,), jnp.float32),
    scratch_types=[
        pltpu.VMEM((FT1 + 1, W1), jnp.int32),
        pltpu.VMEM((FT1 + 1, W1), jnp.float32),
        pltpu.VMEM_SHARED((ND,), jnp.float32),
        pltpu.SemaphoreType.DMA,
        pltpu.SemaphoreType.DMA,
        pltpu.SemaphoreType.DMA,
    ],
)
def _stage1(xf_hbm, xv_hbm, z_hbm, out_hbm, idx_all, vals_all, xd_sh,
            s0, s1, s2):
    cid, sid, gwid = _worker_ids()
    cbase = gwid * FT1 + jnp.minimum(gwid, REM1)

    dz = pltpu.async_copy(z_hbm, xd_sh.at[pl.ds(sid * SL1, SL1)], s1)
    _bulk_load([(xf_hbm, idx_all), (xv_hbm, vals_all)], cbase, gwid,
               FT1, REM1, (s0, s2))
    dz.wait()
    plsc.subcore_barrier()

    @pl.loop(0, FT1)
    def _(t):
        pltpu.sync_copy(vals_all.at[t], xd_sh.at[idx_all.at[t]], add=True)

    @pl.when(gwid < REM1)
    def _():
        pltpu.sync_copy(vals_all.at[FT1], xd_sh.at[idx_all.at[FT1]],
                        add=True)

    plsc.subcore_barrier()
    pltpu.sync_copy(xd_sh.at[pl.ds(sid * SL1, SL1)],
                    out_hbm.at[pl.ds(cid * ND + sid * SL1, SL1)])


@functools.partial(
    pl.kernel,
    mesh=_mesh,
    compiler_params=_cp,
    out_type=jax.ShapeDtypeStruct((NC * N, OUT), jnp.float32),
    scratch_types=[
        pltpu.VMEM((FULL_T + 1, 128), jnp.int32),
        pltpu.VMEM((FULL_T + 1, 128), jnp.int32),
        pltpu.VMEM((FULL_T + 1, 128), jnp.float32),
        pltpu.VMEM((3, 128, OUT), jnp.float32),
        pltpu.VMEM_SHARED((N, OUT), jnp.float32),
        pltpu.SemaphoreType.DMA,
        pltpu.SemaphoreType.DMA,
        pltpu.SemaphoreType.DMA,
        pltpu.SemaphoreType.DMA,
        pltpu.SemaphoreType.DMA,
        pltpu.SemaphoreType.DMA,
    ],
)
def _stage2(ar_hbm, ac_hbm, av_hbm, h_hbm, z_hbm, out_hbm,
            rows_all, cols_all, vals_all, bufs, y_sh,
            g0, g1, g2, s0, s1, s2):
    cid, sid, gwid = _worker_ids()
    cbase = gwid * FULL_T + jnp.minimum(gwid, REM)

    @pl.when(sid < 15)
    def _():
        pltpu.async_copy(z_hbm.at[pl.ds(0, 624)],
                         y_sh.at[pl.ds(sid * 624, 624)], s0)

    @pl.when(sid == 15)
    def _():
        pltpu.async_copy(z_hbm, y_sh.at[pl.ds(15 * 624, 640)], s0)

    _bulk_load([(ar_hbm, rows_all), (ac_hbm, cols_all), (av_hbm, vals_all)],
               cbase, gwid, FULL_T, REM, (g0, g1, g2))

    @pl.when(sid < 15)
    def _():
        pltpu.make_async_copy(z_hbm.at[pl.ds(0, 624)],
                              y_sh.at[pl.ds(sid * 624, 624)], s0).wait()

    @pl.when(sid == 15)
    def _():
        pltpu.make_async_copy(z_hbm, y_sh.at[pl.ds(15 * 624, 640)], s0).wait()

    plsc.subcore_barrier()

    def scale(t, b):
        @pl.loop(0, 8)
        def _(g):
            v16 = vals_all[t, pl.ds(g * 16, 16)]
            for j in range(16):
                k = g * 16 + j
                bc = jnp.full((16,), v16[j], jnp.float32)
                bufs[b, k, pl.ds(0, 16)] = bufs[b, k, pl.ds(0, 16)] * bc
                bufs[b, k, pl.ds(16, 16)] = bufs[b, k, pl.ds(16, 16)] * bc

    nchunks = jnp.where(gwid < REM, FULL_T + 1, FULL_T)
    gsem = (g0, g1, g2)
    ssem = (s0, s1, s2)

    pltpu.async_copy(h_hbm.at[cols_all.at[0]], bufs.at[0], g0)
    pltpu.async_copy(h_hbm.at[cols_all.at[1]], bufs.at[1], g1)

    def chunk_body(c, b):
        bn = (b + 2) % 3
        pltpu.make_async_copy(h_hbm.at[cols_all.at[c]], bufs.at[b],
                              gsem[b]).wait()
        scale(c, b)

        @pl.when(c > 0)
        def _():
            pltpu.make_async_copy(bufs.at[bn], y_sh.at[rows_all.at[c - 1]],
                                  ssem[bn]).wait()

        @pl.when(c + 2 < nchunks)
        def _():
            pltpu.async_copy(h_hbm.at[cols_all.at[c + 2]], bufs.at[bn],
                             gsem[bn])

        pltpu.async_copy(bufs.at[b], y_sh.at[rows_all.at[c]], ssem[b],
                         add=True)

    @pl.loop(0, FULL_T, step=3)
    def _(t):
        chunk_body(t, 0)
        chunk_body(t + 1, 1)
        chunk_body(t + 2, 2)

    pltpu.make_async_copy(bufs.at[2], y_sh.at[rows_all.at[FULL_T - 1]],
                          ssem[2]).wait()

    @pl.when(gwid < REM)
    def _():
        pltpu.make_async_copy(h_hbm.at[cols_all.at[FULL_T]], bufs.at[0],
                              gsem[0]).wait()
        scale(FULL_T, 0)
        pltpu.sync_copy(bufs.at[0], y_sh.at[rows_all.at[FULL_T]], add=True)

    plsc.subcore_barrier()

    @pl.when(sid < 15)
    def _():
        pltpu.sync_copy(y_sh.at[pl.ds(sid * 624, 624)],
                        out_hbm.at[pl.ds(cid * N + sid * 624, 624)])

    @pl.when(sid == 15)
    def _():
        pltpu.sync_copy(y_sh.at[pl.ds(15 * 624, 640)],
                        out_hbm.at[pl.ds(cid * N + 15 * 624, 640)])


def _mm_body(x0_ref, x1_ref, w_ref, h_ref):
    bn = h_ref.shape[0]
    x = (x0_ref[...] + x1_ref[...]).reshape(bn, D)
    h_ref[...] = jnp.dot(x, w_ref[...], preferred_element_type=jnp.float32)


def _matmul(xp, w):
    bn = 2000
    nb = N // bn
    return pl.pallas_call(
        _mm_body,
        grid=(nb,),
        in_specs=[
            pl.BlockSpec((bn * D,), lambda i: (i,)),
            pl.BlockSpec((bn * D,), lambda i: (i + nb,)),
            pl.BlockSpec((D, OUT), lambda i: (0, 0)),
        ],
        out_specs=pl.BlockSpec((bn, OUT), lambda i: (i, 0)),
        out_shape=jax.ShapeDtypeStruct((N, OUT), jnp.float32),
    )(xp, xp, w)


def _fin_body(y0_ref, y1_ref, o_ref):
    o_ref[...] = jnp.maximum(y0_ref[...] + y1_ref[...], 0.0)


def _finish(yp):
    bn = 2000
    nb = N // bn
    return pl.pallas_call(
        _fin_body,
        grid=(nb,),
        in_specs=[
            pl.BlockSpec((bn, OUT), lambda i: (i, 0)),
            pl.BlockSpec((bn, OUT), lambda i: (i + nb, 0)),
        ],
        out_specs=pl.BlockSpec((bn, OUT), lambda i: (i, 0)),
        out_shape=jax.ShapeDtypeStruct((N, OUT), jnp.float32),
    )(yp, yp)


def kernel(x_rows, x_cols, x_vals, adj_rows, adj_cols, adj_vals, kernel):
    xf = (x_rows.astype(jnp.int32) * D
          + x_cols.astype(jnp.int32)).reshape(ROWS1, W1)
    xv = x_vals.reshape(ROWS1, W1)
    ar = adj_rows.astype(jnp.int32).reshape(ROWS, 128)
    ac = adj_cols.astype(jnp.int32).reshape(ROWS, 128)
    av = adj_vals.reshape(ROWS, 128)
    z1 = jnp.zeros((SL1,), jnp.float32)
    z2 = jnp.zeros((640, OUT), jnp.float32)

    xd = _stage1(xf, xv, z1)
    h = _matmul(xd, kernel)
    yp = _stage2(ar, ac, av, h, z2)
    return _finish(yp)
